# baseline (speedup 1.0000x reference)
import functools

import jax
import jax.numpy as jnp
from jax.experimental import pallas as pl
from jax.experimental.pallas import tpu as pltpu

_N = 100000
_D = 128
_BLK = 1000


def _l0_body(x_ref, w_ref, b_ref, deg_ref, o_ref):
    dinv = jax.lax.rsqrt(deg_ref[...])
    h = jnp.dot(x_ref[...], w_ref[...], preferred_element_type=jnp.float32)
    o_ref[...] = (h + b_ref[...]) * dinv


def _mid_body(s_ref, g_ref, deg_ref, w_ref, b_ref, o_ref):
    dinv = jax.lax.rsqrt(deg_ref[...])
    h_in = jnp.maximum(dinv * (s_ref[...] + g_ref[...]), 0.0)
    h = jnp.dot(h_in, w_ref[...], preferred_element_type=jnp.float32)
    o_ref[...] = (h + b_ref[...]) * dinv


def _fin_body(s_ref, g_ref, deg_ref, o_ref):
    dinv = jax.lax.rsqrt(deg_ref[...])
    o_ref[...] = dinv * (s_ref[...] + g_ref[...])


def _row_spec(width):
    return pl.BlockSpec((_BLK, width), lambda i: (i, 0))


def _full_spec(shape):
    return pl.BlockSpec(shape, lambda i: (0, 0))


def _layer0(X, W, b, deg):
    return pl.pallas_call(
        _l0_body,
        grid=(_N // _BLK,),
        in_specs=[
            _row_spec(_D),
            _full_spec(W.shape),
            _full_spec((1, W.shape[1])),
            _row_spec(1),
        ],
        out_specs=_row_spec(W.shape[1]),
        out_shape=jax.ShapeDtypeStruct((_N, W.shape[1]), jnp.float32),
    )(X, W, b.reshape(1, -1), deg)


def _layer_mid(S, G, deg, W, b):
    return pl.pallas_call(
        _mid_body,
        grid=(_N // _BLK,),
        in_specs=[
            _row_spec(_D),
            _row_spec(_D),
            _row_spec(1),
            _full_spec(W.shape),
            _full_spec((1, W.shape[1])),
        ],
        out_specs=_row_spec(W.shape[1]),
        out_shape=jax.ShapeDtypeStruct((_N, W.shape[1]), jnp.float32),
    )(S, G, deg, W, b.reshape(1, -1))


def _layer_fin(S, G, deg):
    width = G.shape[1]
    return pl.pallas_call(
        _fin_body,
        grid=(_N // _BLK,),
        in_specs=[_row_spec(width), _row_spec(width), _row_spec(1)],
        out_specs=_row_spec(width),
        out_shape=jax.ShapeDtypeStruct((_N, width), jnp.float32),
    )(S, G, deg)


def kernel(X, hyperedges, W0, b0, W1, b1, W2, b2):
    he = hyperedges.astype(jnp.int32)
    E, K = he.shape

    Xe = X[he]
    sq = jnp.sum(Xe * Xe, axis=-1)
    gram = jnp.einsum('ekd,emd->ekm', Xe, Xe)
    dist = sq[:, :, None] + sq[:, None, :] - 2.0 * gram
    flat = jnp.argmax(dist.reshape(E, K * K), axis=1)
    i = flat // K
    j = flat % K
    ar = jnp.arange(E)
    u = he[ar, i]
    v = he[ar, j]
    src = jnp.concatenate([u, v])
    dst = jnp.concatenate([v, u])

    cnt = jnp.zeros((_N,), dtype=jnp.float32).at[dst].add(1.0)
    deg = (cnt + 1.0).reshape(_N, 1)

    G0 = _layer0(X, W0, b0, deg)
    S0 = jnp.zeros_like(G0).at[dst].add(G0[src])
    G1 = _layer_mid(S0, G0, deg, W1, b1)
    S1 = jnp.zeros_like(G1).at[dst].add(G1[src])
    G2 = _layer_mid(S1, G1, deg, W2, b2)
    S2 = jnp.zeros_like(G2).at[dst].add(G2[src])
    return _layer_fin(S2, G2, deg)


# speedup vs baseline: 1.0458x; 1.0458x over previous
import functools

import jax
import jax.numpy as jnp
from jax import lax
from jax.experimental import pallas as pl
from jax.experimental.pallas import tpu as pltpu
from jax.experimental.pallas import tpu_sc as plsc

_N = 100000
_D = 128
_BLK = 1000

_NB = 8
_ROWS = 12544
_NPAD = _NB * _ROWS
_EP = 100352
_EW = _EP // 16
_ECH = 1568
_CAP = _EW + 64
_FIRE = 64
_SENT = 0x3FFFFFFF


def _smooth_body(width, g_ref, src_ref, dst_ref, s_ref,
                 esrc, edst, gsrc, grel, gidx, sidx, rows, zrows, acc):
    c = lax.axis_index("c")
    s = lax.axis_index("s")
    nchunk = width // 16
    zero16 = jnp.zeros((16,), jnp.float32)
    iota16 = lax.iota(jnp.int32, 16)

    def _zb(r, carry):
        for t in range(nchunk):
            zrows[r, pl.ds(t * 16, 16)] = zero16
        return carry
    lax.fori_loop(0, 16, _zb, 0)

    eoff = s * _EW
    rows_per_sub = _ROWS // 16
    for blk in range(_NB // 2):
        lo = (c * (_NB // 2) + blk) * _ROWS

        def _zero(z, carry):
            pltpu.sync_copy(zrows, acc.at[pl.ds(s * rows_per_sub + z * 16, 16)])
            return carry
        lax.fori_loop(0, rows_per_sub // 16, _zero, 0)
        plsc.subcore_barrier()

        def _scan_chunk(ch, cnt):
            pltpu.sync_copy(src_ref.at[pl.ds(eoff + ch * _ECH, _ECH)], esrc)
            pltpu.sync_copy(dst_ref.at[pl.ds(eoff + ch * _ECH, _ECH)], edst)

            def _scan(k, cnt):
                base = k * 16
                dvec = edst[pl.ds(base, 16)]
                svec = esrc[pl.ds(base, 16)]
                rel = dvec - lo
                m = (rel >= 0) & (rel < _ROWS)
                mi = m.astype(jnp.int32)
                pos = cnt + plsc.cumsum(mi) - 1
                plsc.store_scatter(gsrc, [pos], svec, mask=m)
                plsc.store_scatter(grel, [pos], rel, mask=m)
                return cnt + jnp.sum(mi)
            return lax.fori_loop(0, _ECH // 16, _scan, cnt)
        cnt = lax.fori_loop(0, _EW // _ECH, _scan_chunk, jnp.int32(0))

        for t in range(4):
            idx = cnt + t * 16 + iota16
            plsc.store_scatter(gsrc, [idx], jnp.zeros((16,), jnp.int32))
            plsc.store_scatter(grel, [idx], jnp.full((16,), _ROWS, jnp.int32))
        nf = (cnt + (_FIRE - 1)) // _FIRE

        def _fire(f, carry):
            for t in range(4):
                gidx[pl.ds(t * 16, 16)] = gsrc[pl.ds(f * _FIRE + t * 16, 16)]
                sidx[pl.ds(t * 16, 16)] = grel[pl.ds(f * _FIRE + t * 16, 16)]
            pltpu.sync_copy(g_ref.at[gidx], rows)
            pltpu.sync_copy(rows, acc.at[sidx], add=True)
            return carry
        lax.fori_loop(0, nf, _fire, 0)
        plsc.subcore_barrier()

        for z in range(7):
            r0 = s * rows_per_sub + z * 112
            pltpu.sync_copy(acc.at[pl.ds(r0, 112)], s_ref.at[pl.ds(lo + r0, 112)])
        plsc.subcore_barrier()


def _smooth(G, src, dst):
    width = G.shape[1]
    mesh = plsc.VectorSubcoreMesh(core_axis_name="c", subcore_axis_name="s")
    kern = pl.kernel(
        functools.partial(_smooth_body, width),
        out_type=jax.ShapeDtypeStruct((_NPAD, width), jnp.float32),
        mesh=mesh,
        compiler_params=pltpu.CompilerParams(needs_layout_passes=False),
        scratch_types=[
            pltpu.VMEM((_ECH,), jnp.int32),
            pltpu.VMEM((_ECH,), jnp.int32),
            pltpu.VMEM((_CAP,), jnp.int32),
            pltpu.VMEM((_CAP,), jnp.int32),
            pltpu.VMEM((_FIRE,), jnp.int32),
            pltpu.VMEM((_FIRE,), jnp.int32),
            pltpu.VMEM((_FIRE, width), jnp.float32),
            pltpu.VMEM((16, width), jnp.float32),
            pltpu.VMEM_SHARED((_ROWS + 8, width), jnp.float32),
        ],
    )
    return kern(G, src, dst)


def _l0_body(x_ref, w_ref, b_ref, deg_ref, o_ref):
    dinv = jax.lax.rsqrt(deg_ref[...])
    h = jnp.dot(x_ref[...], w_ref[...], preferred_element_type=jnp.float32)
    o_ref[...] = (h + b_ref[...]) * dinv


def _mid_body(s_ref, g_ref, deg_ref, w_ref, b_ref, o_ref):
    dinv = jax.lax.rsqrt(deg_ref[...])
    h_in = jnp.maximum(dinv * (s_ref[...] + g_ref[...]), 0.0)
    h = jnp.dot(h_in, w_ref[...], preferred_element_type=jnp.float32)
    o_ref[...] = (h + b_ref[...]) * dinv


def _fin_body(s_ref, g_ref, deg_ref, o_ref):
    dinv = jax.lax.rsqrt(deg_ref[...])
    o_ref[...] = dinv * (s_ref[...] + g_ref[...])


def _row_spec(width):
    return pl.BlockSpec((_BLK, width), lambda i: (i, 0))


def _full_spec(shape):
    return pl.BlockSpec(shape, lambda i: (0, 0))


def _layer0(X, W, b, deg):
    return pl.pallas_call(
        _l0_body,
        grid=(_N // _BLK,),
        in_specs=[
            _row_spec(_D),
            _full_spec(W.shape),
            _full_spec((1, W.shape[1])),
            _row_spec(1),
        ],
        out_specs=_row_spec(W.shape[1]),
        out_shape=jax.ShapeDtypeStruct((_N, W.shape[1]), jnp.float32),
    )(X, W, b.reshape(1, -1), deg)


def _layer_mid(S, G, deg, W, b):
    return pl.pallas_call(
        _mid_body,
        grid=(_N // _BLK,),
        in_specs=[
            _row_spec(_D),
            _row_spec(_D),
            _row_spec(1),
            _full_spec(W.shape),
            _full_spec((1, W.shape[1])),
        ],
        out_specs=_row_spec(W.shape[1]),
        out_shape=jax.ShapeDtypeStruct((_N, W.shape[1]), jnp.float32),
    )(S, G, deg, W, b.reshape(1, -1))


def _layer_fin(S, G, deg):
    width = G.shape[1]
    return pl.pallas_call(
        _fin_body,
        grid=(_N // _BLK,),
        in_specs=[_row_spec(width), _row_spec(width), _row_spec(1)],
        out_specs=_row_spec(width),
        out_shape=jax.ShapeDtypeStruct((_N, width), jnp.float32),
    )(S, G, deg)


def kernel(X, hyperedges, W0, b0, W1, b1, W2, b2):
    he = hyperedges.astype(jnp.int32)
    E, K = he.shape

    Xe = X[he]
    sq = jnp.sum(Xe * Xe, axis=-1)
    gram = jnp.einsum('ekd,emd->ekm', Xe, Xe)
    dist = sq[:, :, None] + sq[:, None, :] - 2.0 * gram
    flat = jnp.argmax(dist.reshape(E, K * K), axis=1)
    i = flat // K
    j = flat % K
    ar = jnp.arange(E)
    u = he[ar, i]
    v = he[ar, j]
    src = jnp.concatenate([u, v])
    dst = jnp.concatenate([v, u])

    cnt = jnp.zeros((_N,), dtype=jnp.float32).at[dst].add(1.0)
    deg = (cnt + 1.0).reshape(_N, 1)

    npad = _EP - src.shape[0]
    src_p = jnp.concatenate([src, jnp.zeros((npad,), jnp.int32)])
    dst_p = jnp.concatenate([dst, jnp.full((npad,), _SENT, jnp.int32)])

    W2p = jnp.pad(W2, ((0, 0), (0, 88)))
    b2p = jnp.pad(b2, (0, 88))

    G0 = _layer0(X, W0, b0, deg)
    S0 = _smooth(G0, src_p, dst_p)[:_N]
    G1 = _layer_mid(S0, G0, deg, W1, b1)
    S1 = _smooth(G1, src_p, dst_p)[:_N]
    G2 = _layer_mid(S1, G1, deg, W2p, b2p)
    S2 = _smooth(G2, src_p, dst_p)[:_N]
    return _layer_fin(S2, G2, deg)[:, :40]
